# Initial kernel scaffold; baseline (speedup 1.0000x reference)
import jax
import jax.numpy as jnp
from jax.experimental import pallas as pl


def kernel(times_in, times_out, segment_filter_ids, one_hot_predecessor_ids, decay_rate):
    raise NotImplementedError("write your pallas kernel here")


# speedup vs baseline: 17.8723x; 17.8723x over previous
import functools

import jax
import jax.numpy as jnp
from jax import lax
from jax.experimental import pallas as pl
from jax.experimental.pallas import tpu as pltpu
from jax.experimental.pallas import tpu_sc as plsc

E = 1_600_000
S = 50_000
F = 16
NW = 32
PS = 1_568
S_PAD = NW * PS
C = 1_024
LOG2C = 10


def _sc_body(tin_hbm, tout_hbm, seg_hbm, oh_hbm, nrate_hbm, bounds_hbm,
             out_hbm, tout_v, num_v, den_v, seg_v, tin_v, oh_v, nrate_v,
             bounds_v):
    wid = lax.axis_index("c") * 16 + lax.axis_index("s")
    seg_base = wid * PS

    pltpu.sync_copy(bounds_hbm, bounds_v)
    pltpu.sync_copy(nrate_hbm, nrate_v)
    pltpu.sync_copy(tout_hbm.at[pl.ds(seg_base, PS)], tout_v.at[pl.ds(0, PS)])
    tout_v[pl.ds(PS, 16)] = jnp.zeros((16,), jnp.float32)

    zeros16 = jnp.zeros((16,), jnp.float32)

    def zero_body(i, _):
        num_v[pl.ds(i * 16, 16)] = zeros16
        den_v[pl.ds(i * 16, 16)] = zeros16
        return 0
    lax.fori_loop(0, PS + 1, zero_body, 0)

    bvec = bounds_v[pl.ds(wid, 16)]
    a = bvec[0]
    a_end = bvec[1]
    b = lax.bitwise_and(a, -8)
    nk = lax.shift_right_logical(a_end - b + (C - 1), LOG2C)

    nrate = nrate_v[...]
    iota16 = lax.broadcasted_iota(jnp.int32, (16,), 0)
    segb_splat = jnp.full((16,), seg_base, jnp.int32)
    ps_splat = jnp.full((16,), PS, jnp.int32)
    aend_splat = jnp.full((16,), a_end, jnp.int32)

    def chunk_body(k, _):
        start = b + lax.shift_left(k, LOG2C)
        e0 = pl.multiple_of(jnp.minimum(start, E - C), 8)
        lo = jnp.maximum(a, start)
        pltpu.sync_copy(seg_hbm.at[pl.ds(e0, C)], seg_v)
        pltpu.sync_copy(tin_hbm.at[pl.ds(e0, C)], tin_v)
        pltpu.sync_copy(
            oh_hbm.at[pl.ds(pl.multiple_of(lax.shift_left(e0, 4), 8), C * F)],
            oh_v)
        lo_splat = jnp.full((16,), lo, jnp.int32)

        def group_body(g, _):
            gbase = lax.shift_left(g, 4)
            segv = seg_v[pl.ds(gbase, 16)]
            tinv = tin_v[pl.ds(gbase, 16)]
            gvv = jnp.full((16,), e0 + gbase, jnp.int32) + iota16
            validv = jnp.logical_and(gvv >= lo_splat, gvv < aend_splat)
            slv = jnp.where(validv, segv - segb_splat, ps_splat)
            toutv = plsc.load_gather(tout_v, [slv])
            dtv = toutv - tinv
            offv = lax.shift_left(slv, 4)
            rowbase = lax.shift_left(g, 8)
            for u in range(16):
                ev = jnp.exp(nrate * lax.broadcast_in_dim(dtv[u], (16,), ()))
                row = oh_v[pl.ds(rowbase + u * 16, 16)]
                off = offv[u]
                plsc.addupdate(num_v.at[pl.ds(off, 16)], row * ev)
                plsc.addupdate(den_v.at[pl.ds(off, 16)], row)
            return 0
        lax.fori_loop(0, C // 16, group_body, 0)
        return 0
    lax.fori_loop(0, nk, chunk_body, 0)

    ones16 = jnp.ones((16,), jnp.float32)

    def div_body(i, _):
        o = i * 16
        num_v[pl.ds(o, 16)] = num_v[pl.ds(o, 16)] / jnp.maximum(
            den_v[pl.ds(o, 16)], ones16)
        return 0
    lax.fori_loop(0, PS, div_body, 0)

    pltpu.sync_copy(num_v.at[pl.ds(0, PS * F)],
                    out_hbm.at[pl.ds(seg_base * F, PS * F)])


@jax.jit
def _run(times_in, tout_pad, segment_filter_ids, oh_flat, nrate, bounds):
    mesh = plsc.VectorSubcoreMesh(core_axis_name="c", subcore_axis_name="s")
    f = pl.kernel(
        _sc_body,
        out_type=jax.ShapeDtypeStruct((S_PAD * F,), jnp.float32),
        mesh=mesh,
        scratch_types=[
            pltpu.VMEM((PS + 16,), jnp.float32),
            pltpu.VMEM(((PS + 1) * F,), jnp.float32),
            pltpu.VMEM(((PS + 1) * F,), jnp.float32),
            pltpu.VMEM((C,), jnp.int32),
            pltpu.VMEM((C,), jnp.float32),
            pltpu.VMEM((C * F,), jnp.float32),
            pltpu.VMEM((16,), jnp.float32),
            pltpu.VMEM((48,), jnp.int32),
        ],
        compiler_params=pltpu.CompilerParams(needs_layout_passes=False),
    )
    return f(times_in, tout_pad, segment_filter_ids, oh_flat, nrate, bounds)


def kernel(times_in, times_out, segment_filter_ids, one_hot_predecessor_ids,
           decay_rate):
    nrate = -jax.nn.softplus(decay_rate)
    tout_pad = jnp.pad(times_out, (0, S_PAD - S))
    limits = jnp.minimum(jnp.arange(NW + 1, dtype=jnp.int32) * PS, S)
    bounds = jnp.searchsorted(segment_filter_ids, limits, side="left",
                              method="scan_unrolled").astype(jnp.int32)
    bounds = jnp.pad(bounds, (0, 48 - (NW + 1)))
    oh_flat = one_hot_predecessor_ids.reshape(E * F)
    out = _run(times_in, tout_pad, segment_filter_ids, oh_flat, nrate, bounds)
    return out.reshape(S_PAD, F)[:S]


# speedup vs baseline: 19.9871x; 1.1183x over previous
import functools

import jax
import jax.numpy as jnp
from jax import lax
from jax.experimental import pallas as pl
from jax.experimental.pallas import tpu as pltpu
from jax.experimental.pallas import tpu_sc as plsc

E = 1_600_000
S = 50_000
F = 16
NW = 32
PS = 1_568
S_PAD = NW * PS
C = 1_024
LOG2C = 10


def _sc_body(tin_hbm, tout_hbm, seg_hbm, oh_hbm, nrate_hbm, bounds_hbm,
             out_hbm, tout_v, num_v, den_v, seg_v, tin_v, oh_v, nrate_v,
             bounds_v):
    wid = lax.axis_index("c") * 16 + lax.axis_index("s")
    seg_base = wid * PS

    pltpu.sync_copy(bounds_hbm, bounds_v)
    pltpu.sync_copy(nrate_hbm, nrate_v)
    pltpu.sync_copy(tout_hbm.at[pl.ds(seg_base, PS)], tout_v.at[pl.ds(0, PS)])
    tout_v[pl.ds(PS, 16)] = jnp.zeros((16,), jnp.float32)

    zeros16 = jnp.zeros((16,), jnp.float32)

    def zero_body(i, _):
        num_v[pl.ds(i * 16, 16)] = zeros16
        den_v[pl.ds(i * 16, 16)] = zeros16
        return 0
    lax.fori_loop(0, PS + 1, zero_body, 0)

    bvec = bounds_v[pl.ds(wid, 16)]
    a = bvec[0]
    a_end = bvec[1]
    b = lax.bitwise_and(a, -8)
    nk = lax.shift_right_logical(a_end - b + (C - 1), LOG2C)

    nrate = nrate_v[...]
    iota16 = lax.broadcasted_iota(jnp.int32, (16,), 0)
    iotax16 = iota16 * 16
    segb_splat = jnp.full((16,), seg_base, jnp.int32)
    ps_splat = jnp.full((16,), PS, jnp.int32)
    aend_splat = jnp.full((16,), a_end, jnp.int32)
    ones16f = jnp.ones((16,), jnp.float32)

    def chunk_body(k, _):
        start = b + lax.shift_left(k, LOG2C)
        e0 = pl.multiple_of(jnp.minimum(start, E - C), 8)
        lo = jnp.maximum(a, start)
        pltpu.sync_copy(seg_hbm.at[pl.ds(e0, C)], seg_v)
        pltpu.sync_copy(tin_hbm.at[pl.ds(e0, C)], tin_v)
        pltpu.sync_copy(
            oh_hbm.at[pl.ds(pl.multiple_of(lax.shift_left(e0, 4), 8), C * F)],
            oh_v)
        lo_splat = jnp.full((16,), lo, jnp.int32)

        def group_body(g, _):
            gbase = lax.shift_left(g, 4)
            segv = seg_v[pl.ds(gbase, 16)]
            tinv = tin_v[pl.ds(gbase, 16)]
            gvv = jnp.full((16,), e0 + gbase, jnp.int32) + iota16
            validv = jnp.logical_and(gvv >= lo_splat, gvv < aend_splat)
            slv = jnp.where(validv, segv - segb_splat, ps_splat)
            toutv = plsc.load_gather(tout_v, [slv])
            dtv = toutv - tinv
            bidx = jnp.full((16,), lax.shift_left(g, 8), jnp.int32) + iotax16
            terms = [
                plsc.load_gather(oh_v, [bidx + f]) * jnp.float32(f)
                for f in range(1, 16)
            ]
            while len(terms) > 1:
                terms = [terms[i] + terms[i + 1]
                         for i in range(0, len(terms) - 1, 2)] + (
                             [terms[-1]] if len(terms) % 2 else [])
            predv = terms[0].astype(jnp.int32)
            ratev = plsc.load_gather(nrate_v, [predv])
            valv = jnp.exp(ratev * dtv)
            idxv = lax.shift_left(slv, 4) + predv
            plsc.addupdate_scatter(num_v, [idxv], valv, mask=validv)
            plsc.addupdate_scatter(den_v, [idxv], ones16f, mask=validv)
            return 0
        lax.fori_loop(0, C // 16, group_body, 0)
        return 0
    lax.fori_loop(0, nk, chunk_body, 0)

    ones16 = jnp.ones((16,), jnp.float32)

    def div_body(i, _):
        o = i * 16
        num_v[pl.ds(o, 16)] = num_v[pl.ds(o, 16)] / jnp.maximum(
            den_v[pl.ds(o, 16)], ones16)
        return 0
    lax.fori_loop(0, PS, div_body, 0)

    pltpu.sync_copy(num_v.at[pl.ds(0, PS * F)],
                    out_hbm.at[pl.ds(seg_base * F, PS * F)])


@jax.jit
def _run(times_in, tout_pad, segment_filter_ids, oh_flat, nrate, bounds):
    mesh = plsc.VectorSubcoreMesh(core_axis_name="c", subcore_axis_name="s")
    f = pl.kernel(
        _sc_body,
        out_type=jax.ShapeDtypeStruct((S_PAD * F,), jnp.float32),
        mesh=mesh,
        scratch_types=[
            pltpu.VMEM((PS + 16,), jnp.float32),
            pltpu.VMEM(((PS + 1) * F,), jnp.float32),
            pltpu.VMEM(((PS + 1) * F,), jnp.float32),
            pltpu.VMEM((C,), jnp.int32),
            pltpu.VMEM((C,), jnp.float32),
            pltpu.VMEM((C * F,), jnp.float32),
            pltpu.VMEM((16,), jnp.float32),
            pltpu.VMEM((48,), jnp.int32),
        ],
        compiler_params=pltpu.CompilerParams(needs_layout_passes=False),
    )
    return f(times_in, tout_pad, segment_filter_ids, oh_flat, nrate, bounds)


def kernel(times_in, times_out, segment_filter_ids, one_hot_predecessor_ids,
           decay_rate):
    nrate = -jax.nn.softplus(decay_rate)
    tout_pad = jnp.pad(times_out, (0, S_PAD - S))
    limits = jnp.minimum(jnp.arange(NW + 1, dtype=jnp.int32) * PS, S)
    bounds = jnp.searchsorted(segment_filter_ids, limits, side="left",
                              method="scan_unrolled").astype(jnp.int32)
    bounds = jnp.pad(bounds, (0, 48 - (NW + 1)))
    oh_flat = one_hot_predecessor_ids.reshape(E * F)
    out = _run(times_in, tout_pad, segment_filter_ids, oh_flat, nrate, bounds)
    return out.reshape(S_PAD, F)[:S]


# speedup vs baseline: 22.7167x; 1.1366x over previous
import functools

import jax
import jax.numpy as jnp
from jax import lax
from jax.experimental import pallas as pl
from jax.experimental.pallas import tpu as pltpu
from jax.experimental.pallas import tpu_sc as plsc

E = 1_600_000
S = 50_000
F = 16
NW = 32
PS = 1_568
S_PAD = NW * PS
C = 1_024
LOG2C = 10


def _sc_body(tin_hbm, tout_hbm, seg_hbm, oh_hbm, nrate_hbm, bounds_hbm,
             out_hbm, tout_v, num_v, den_v, seg_v, tin_v, oh_v, nrate_v,
             bounds_v, sem0, sem1):
    wid = lax.axis_index("c") * 16 + lax.axis_index("s")
    seg_base = wid * PS

    pltpu.sync_copy(bounds_hbm, bounds_v)
    pltpu.sync_copy(nrate_hbm, nrate_v)
    pltpu.sync_copy(tout_hbm.at[pl.ds(seg_base, PS)], tout_v.at[pl.ds(0, PS)])
    tout_v[pl.ds(PS, 16)] = jnp.zeros((16,), jnp.float32)

    zeros16 = jnp.zeros((16,), jnp.float32)

    def zero_body(i, _):
        num_v[pl.ds(i * 16, 16)] = zeros16
        den_v[pl.ds(i * 16, 16)] = zeros16
        return 0
    lax.fori_loop(0, PS + 1, zero_body, 0)

    bvec = bounds_v[pl.ds(wid, 16)]
    a = bvec[0]
    a_end = bvec[1]
    b = lax.bitwise_and(a, -8)
    nk = lax.shift_right_logical(a_end - b + (C - 1), LOG2C)

    nrate = nrate_v[...]
    iota16 = lax.broadcasted_iota(jnp.int32, (16,), 0)
    iotax16 = iota16 * 16
    segb_splat = jnp.full((16,), seg_base, jnp.int32)
    ps_splat = jnp.full((16,), PS, jnp.int32)
    aend_splat = jnp.full((16,), a_end, jnp.int32)
    ones16f = jnp.ones((16,), jnp.float32)
    step16 = jnp.full((16,), 16, jnp.int32)
    step256 = jnp.full((16,), 256, jnp.int32)

    def dma_start(k, slot):
        start = b + lax.shift_left(k, LOG2C)
        e0 = pl.multiple_of(jnp.minimum(start, E - C), 8)
        sb = slot * C
        sem = sem0 if slot == 0 else sem1
        pltpu.async_copy(seg_hbm.at[pl.ds(e0, C)], seg_v.at[pl.ds(sb, C)],
                         sem)
        pltpu.async_copy(tin_hbm.at[pl.ds(e0, C)], tin_v.at[pl.ds(sb, C)],
                         sem)
        pltpu.async_copy(
            oh_hbm.at[pl.ds(pl.multiple_of(lax.shift_left(e0, 4), 8), C * F)],
            oh_v.at[pl.ds(sb * F, C * F)], sem)

    def dma_wait(slot):
        sb = slot * C
        sem = sem0 if slot == 0 else sem1
        pltpu.make_async_copy(seg_hbm.at[pl.ds(0, C)],
                              seg_v.at[pl.ds(sb, C)], sem).wait()
        pltpu.make_async_copy(tin_hbm.at[pl.ds(0, C)],
                              tin_v.at[pl.ds(sb, C)], sem).wait()
        pltpu.make_async_copy(oh_hbm.at[pl.ds(0, C * F)],
                              oh_v.at[pl.ds(sb * F, C * F)], sem).wait()

    def compute(k, slot):
        start = b + lax.shift_left(k, LOG2C)
        e0 = jnp.minimum(start, E - C)
        lo = jnp.maximum(a, start)
        lo_splat = jnp.full((16,), lo, jnp.int32)
        sb = slot * C
        gvv0 = jnp.full((16,), e0, jnp.int32) + iota16
        bidx0 = jnp.full((16,), sb * F, jnp.int32) + iotax16

        def group_body(g, carry):
            gvv, bidx = carry
            gbase = sb + lax.shift_left(g, 4)
            segv = seg_v[pl.ds(gbase, 16)]
            tinv = tin_v[pl.ds(gbase, 16)]
            validv = jnp.logical_and(gvv >= lo_splat, gvv < aend_splat)
            slv = jnp.where(validv, segv - segb_splat, ps_splat)
            toutv = plsc.load_gather(tout_v, [slv])
            dtv = toutv - tinv
            terms = [
                plsc.load_gather(oh_v, [bidx + f]) * jnp.float32(f)
                for f in range(1, 16)
            ]
            while len(terms) > 1:
                terms = [terms[i] + terms[i + 1]
                         for i in range(0, len(terms) - 1, 2)] + (
                             [terms[-1]] if len(terms) % 2 else [])
            predv = terms[0].astype(jnp.int32)
            ratev = plsc.load_gather(nrate_v, [predv])
            valv = jnp.exp(ratev * dtv)
            idxv = lax.shift_left(slv, 4) + predv
            plsc.addupdate_scatter(num_v, [idxv], valv, mask=validv)
            plsc.addupdate_scatter(den_v, [idxv], ones16f, mask=validv)
            return (gvv + step16, bidx + step256)
        lax.fori_loop(0, C // 16, group_body, (gvv0, bidx0))

    @pl.when(nk > 0)
    def _():
        dma_start(0, 0)

    def pair_body(p, _):
        k0 = lax.shift_left(p, 1)
        k1 = k0 + 1

        @pl.when(k1 < nk)
        def _():
            dma_start(k1, 1)
        dma_wait(0)
        compute(k0, 0)

        @pl.when(k1 + 1 < nk)
        def _():
            dma_start(k1 + 1, 0)

        @pl.when(k1 < nk)
        def _():
            dma_wait(1)
            compute(k1, 1)
        return 0
    lax.fori_loop(0, lax.shift_right_logical(nk + 1, 1), pair_body, 0)

    ones16 = jnp.ones((16,), jnp.float32)

    def div_body(i, _):
        o = i * 16
        num_v[pl.ds(o, 16)] = num_v[pl.ds(o, 16)] / jnp.maximum(
            den_v[pl.ds(o, 16)], ones16)
        return 0
    lax.fori_loop(0, PS, div_body, 0)

    pltpu.sync_copy(num_v.at[pl.ds(0, PS * F)],
                    out_hbm.at[pl.ds(seg_base * F, PS * F)])


@jax.jit
def _run(times_in, tout_pad, segment_filter_ids, oh_flat, nrate, bounds):
    mesh = plsc.VectorSubcoreMesh(core_axis_name="c", subcore_axis_name="s")
    f = pl.kernel(
        _sc_body,
        out_type=jax.ShapeDtypeStruct((S_PAD * F,), jnp.float32),
        mesh=mesh,
        scratch_types=[
            pltpu.VMEM((PS + 16,), jnp.float32),
            pltpu.VMEM(((PS + 1) * F,), jnp.float32),
            pltpu.VMEM(((PS + 1) * F,), jnp.float32),
            pltpu.VMEM((2 * C,), jnp.int32),
            pltpu.VMEM((2 * C,), jnp.float32),
            pltpu.VMEM((2 * C * F,), jnp.float32),
            pltpu.VMEM((16,), jnp.float32),
            pltpu.VMEM((48,), jnp.int32),
            pltpu.SemaphoreType.DMA,
            pltpu.SemaphoreType.DMA,
        ],
        compiler_params=pltpu.CompilerParams(needs_layout_passes=False),
    )
    return f(times_in, tout_pad, segment_filter_ids, oh_flat, nrate, bounds)


def kernel(times_in, times_out, segment_filter_ids, one_hot_predecessor_ids,
           decay_rate):
    nrate = -jax.nn.softplus(decay_rate)
    tout_pad = jnp.pad(times_out, (0, S_PAD - S))
    limits = jnp.minimum(jnp.arange(NW + 1, dtype=jnp.int32) * PS, S)
    bounds = jnp.searchsorted(segment_filter_ids, limits, side="left",
                              method="scan_unrolled").astype(jnp.int32)
    bounds = jnp.pad(bounds, (0, 48 - (NW + 1)))
    oh_flat = one_hot_predecessor_ids.reshape(E * F)
    out = _run(times_in, tout_pad, segment_filter_ids, oh_flat, nrate, bounds)
    return out.reshape(S_PAD, F)[:S]
